# Initial kernel scaffold; baseline (speedup 1.0000x reference)
import jax
import jax.numpy as jnp
from jax.experimental import pallas as pl


def kernel(current_node, encoded_input, siblings, siblings_mask):
    raise NotImplementedError("write your pallas kernel here")


# speedup vs baseline: 1.0646x; 1.0646x over previous
import jax
import jax.numpy as jnp
from jax import lax
from jax.experimental import pallas as pl
from jax.experimental.pallas import tpu as pltpu
from jax.experimental.pallas import tpu_sc as plsc

B, S, D, L = 1024, 512, 256, 50
NC, NS, LANES = 2, 16, 16
NW = NC * NS
RPW = B // NW
GROUPS = RPW // LANES
DV = D // LANES


def _sc_body(cn_hbm, enc_hbm, mask_hbm, sib_hbm, out_hbm,
             mask_v, sib_v, cn_v, rows_v, idx_v, mfac_v, sem):
    wid = lax.axis_index("c") * NS + lax.axis_index("s")
    base = wid * RPW

    pltpu.sync_copy(mask_hbm.at[wid], mask_v)
    pltpu.sync_copy(sib_hbm.at[wid], sib_v)
    pltpu.sync_copy(cn_hbm.at[pl.ds(base, RPW)], cn_v)

    lane = lax.iota(jnp.int32, LANES)
    for g in range(GROUPS):
        sl = pl.ds(g * LANES, LANES)
        n = mask_v[0, sl]
        for l in range(1, L):
            n = n + mask_v[l, sl]
        last = jnp.clip(n - 1, 0, L - 1)
        col = lane + g * LANES
        sib = plsc.load_gather(sib_v, [last * RPW + col])
        idx_v[sl] = (base + col) * S + sib
        mfac_v[sl] = (n != 0).astype(jnp.float32)

    pltpu.async_copy(enc_hbm.at[idx_v], rows_v, sem).wait()

    def combine(r, carry):
        mrow = plsc.load_gather(mfac_v, [jnp.full((LANES,), r, jnp.int32)])
        for d in range(DV):
            dsl = pl.ds(d * LANES, LANES)
            rows_v[r, dsl] = cn_v[r, dsl] + mrow * rows_v[r, dsl]
        return carry

    lax.fori_loop(0, RPW, combine, 0)
    pltpu.sync_copy(rows_v, out_hbm.at[pl.ds(base, RPW)])


def kernel(current_node, encoded_input, siblings, siblings_mask):
    enc2 = encoded_input.reshape(B * S, D)
    mask_w = (siblings_mask.astype(jnp.int32).T
              .reshape(L, NW, RPW).swapaxes(0, 1))
    sib_w = (siblings.astype(jnp.int32).T
             .reshape(L, NW, RPW).swapaxes(0, 1).reshape(NW, L * RPW))

    run = pl.kernel(
        _sc_body,
        out_type=jax.ShapeDtypeStruct((B, D), jnp.float32),
        mesh=plsc.VectorSubcoreMesh(core_axis_name="c", subcore_axis_name="s"),
        compiler_params=pltpu.CompilerParams(needs_layout_passes=False),
        scratch_types=[
            pltpu.VMEM((L, RPW), jnp.int32),
            pltpu.VMEM((L * RPW,), jnp.int32),
            pltpu.VMEM((RPW, D), jnp.float32),
            pltpu.VMEM((RPW, D), jnp.float32),
            pltpu.VMEM((RPW,), jnp.int32),
            pltpu.VMEM((RPW,), jnp.float32),
            pltpu.SemaphoreType.DMA,
        ],
    )
    return run(current_node, enc2, mask_w, sib_w)


# speedup vs baseline: 1.0815x; 1.0159x over previous
import jax
import jax.numpy as jnp
from jax import lax
from jax.experimental import pallas as pl
from jax.experimental.pallas import tpu as pltpu
from jax.experimental.pallas import tpu_sc as plsc

B, S, D, L = 1024, 512, 256, 50
NC, NS, LANES = 2, 16, 16
NW = NC * NS
RPW = B // NW
GROUPS = RPW // LANES
DV = D // LANES


def _sc_body(cn_hbm, enc_hbm, mask_hbm, sib_hbm, out_hbm,
             mask_v, sib_v, cn_v, rows_v, idx_v, mfac_v,
             sem_in, sem_cn, sem_g0, sem_g1, sem_out):
    wid = lax.axis_index("c") * NS + lax.axis_index("s")
    base = wid * RPW

    c_mask = pltpu.async_copy(mask_hbm.at[wid], mask_v, sem_in)
    c_sib = pltpu.async_copy(sib_hbm.at[wid], sib_v, sem_in)
    c_cn = pltpu.async_copy(cn_hbm.at[pl.ds(base, RPW)], cn_v, sem_cn)
    c_mask.wait()
    c_sib.wait()

    lane = lax.iota(jnp.int32, LANES)
    sem_gs = [sem_g0, sem_g1]
    gathers = []
    for g in range(GROUPS):
        sl = pl.ds(g * LANES, LANES)
        n = mask_v[0, sl]
        for l in range(1, L):
            n = n + mask_v[l, sl]
        last = jnp.clip(n - 1, 0, L - 1)
        col = lane + g * LANES
        sib = plsc.load_gather(sib_v, [last * RPW + col])
        idx_v[sl] = (base + col) * S + sib
        mfac_v[sl] = (n != 0).astype(jnp.float32)
        gathers.append(pltpu.async_copy(
            enc_hbm.at[idx_v.at[sl]], rows_v.at[sl], sem_gs[g]))

    c_cn.wait()
    outs = []
    for g in range(GROUPS):
        gathers[g].wait()
        for j in range(LANES):
            r = g * LANES + j
            mrow = plsc.load_gather(mfac_v, [jnp.full((LANES,), r, jnp.int32)])
            for d in range(DV):
                dsl = pl.ds(d * LANES, LANES)
                rows_v[r, dsl] = cn_v[r, dsl] + mrow * rows_v[r, dsl]
        sl = pl.ds(g * LANES, LANES)
        outs.append(pltpu.async_copy(
            rows_v.at[sl], out_hbm.at[pl.ds(base + g * LANES, LANES)], sem_out))
    for c in outs:
        c.wait()


def kernel(current_node, encoded_input, siblings, siblings_mask):
    enc2 = encoded_input.reshape(B * S, D)
    mask_w = (siblings_mask.astype(jnp.int32).T
              .reshape(L, NW, RPW).swapaxes(0, 1))
    sib_w = (siblings.astype(jnp.int32).T
             .reshape(L, NW, RPW).swapaxes(0, 1).reshape(NW, L * RPW))

    run = pl.kernel(
        _sc_body,
        out_type=jax.ShapeDtypeStruct((B, D), jnp.float32),
        mesh=plsc.VectorSubcoreMesh(core_axis_name="c", subcore_axis_name="s"),
        compiler_params=pltpu.CompilerParams(needs_layout_passes=False),
        scratch_types=[
            pltpu.VMEM((L, RPW), jnp.int32),
            pltpu.VMEM((L * RPW,), jnp.int32),
            pltpu.VMEM((RPW, D), jnp.float32),
            pltpu.VMEM((RPW, D), jnp.float32),
            pltpu.VMEM((RPW,), jnp.int32),
            pltpu.VMEM((RPW,), jnp.float32),
            pltpu.SemaphoreType.DMA,
            pltpu.SemaphoreType.DMA,
            pltpu.SemaphoreType.DMA,
            pltpu.SemaphoreType.DMA,
            pltpu.SemaphoreType.DMA,
        ],
    )
    return run(current_node, enc2, mask_w, sib_w)


# speedup vs baseline: 1.1216x; 1.0371x over previous
import jax
import jax.numpy as jnp
from jax import lax
from jax.experimental import pallas as pl
from jax.experimental.pallas import tpu as pltpu
from jax.experimental.pallas import tpu_sc as plsc

B, S, D, L = 1024, 512, 256, 50
NC, NS, LANES = 2, 16, 16
NW = NC * NS
RPW = B // NW
GROUPS = RPW // LANES
DV = D // LANES


def _sc_body(cn_hbm, enc_hbm, mask_hbm, sib_hbm, out_hbm,
             mask_v, sib_v, cn_v, rows_v, idx_v, mfac_v,
             sem_in, sem_cn, sem_g0, sem_g1, sem_out):
    wid = lax.axis_index("c") * NS + lax.axis_index("s")
    base = wid * RPW

    c_mask = pltpu.async_copy(mask_hbm.at[wid], mask_v, sem_in)
    c_sib = pltpu.async_copy(sib_hbm.at[wid], sib_v, sem_in)
    c_cn = pltpu.async_copy(cn_hbm.at[pl.ds(base, RPW)], cn_v, sem_cn)
    c_mask.wait()
    c_sib.wait()

    lane = lax.iota(jnp.int32, LANES)
    sem_gs = [sem_g0, sem_g1]
    gathers = []
    for g in range(GROUPS):
        sl = pl.ds(g * LANES, LANES)
        rowbase = (lane + g * LANES) * L
        n = plsc.load_gather(mask_v, [rowbase])
        for l in range(1, L):
            n = n + plsc.load_gather(mask_v, [rowbase + l])
        last = jnp.clip(n - 1, 0, L - 1)
        sib = plsc.load_gather(sib_v, [rowbase + last])
        idx_v[sl] = (base + lane + g * LANES) * S + sib
        mfac_v[sl] = (n != 0).astype(jnp.float32)
        gathers.append(pltpu.async_copy(
            enc_hbm.at[idx_v.at[sl]], rows_v.at[sl], sem_gs[g]))

    c_cn.wait()
    outs = []
    for g in range(GROUPS):
        gathers[g].wait()
        for j in range(LANES):
            r = g * LANES + j
            mrow = plsc.load_gather(mfac_v, [jnp.full((LANES,), r, jnp.int32)])
            for d in range(DV):
                dsl = pl.ds(d * LANES, LANES)
                rows_v[r, dsl] = cn_v[r, dsl] + mrow * rows_v[r, dsl]
        sl = pl.ds(g * LANES, LANES)
        outs.append(pltpu.async_copy(
            rows_v.at[sl], out_hbm.at[pl.ds(base + g * LANES, LANES)], sem_out))
    for c in outs:
        c.wait()


def kernel(current_node, encoded_input, siblings, siblings_mask):
    enc2 = encoded_input.reshape(B * S, D)
    mask_w = siblings_mask.astype(jnp.int32).reshape(NW, RPW * L)
    sib_w = siblings.astype(jnp.int32).reshape(NW, RPW * L)

    run = pl.kernel(
        _sc_body,
        out_type=jax.ShapeDtypeStruct((B, D), jnp.float32),
        mesh=plsc.VectorSubcoreMesh(core_axis_name="c", subcore_axis_name="s"),
        compiler_params=pltpu.CompilerParams(needs_layout_passes=False),
        scratch_types=[
            pltpu.VMEM((RPW * L,), jnp.int32),
            pltpu.VMEM((RPW * L,), jnp.int32),
            pltpu.VMEM((RPW, D), jnp.float32),
            pltpu.VMEM((RPW, D), jnp.float32),
            pltpu.VMEM((RPW,), jnp.int32),
            pltpu.VMEM((RPW,), jnp.float32),
            pltpu.SemaphoreType.DMA,
            pltpu.SemaphoreType.DMA,
            pltpu.SemaphoreType.DMA,
            pltpu.SemaphoreType.DMA,
            pltpu.SemaphoreType.DMA,
        ],
    )
    return run(current_node, enc2, mask_w, sib_w)
